# Initial kernel scaffold; baseline (speedup 1.0000x reference)
import jax
import jax.numpy as jnp
from jax.experimental import pallas as pl


def kernel(x, edge_index, W1, b1, W2, b2, W3, b3, jk, fcW, fcb):
    raise NotImplementedError("write your pallas kernel here")


# speedup vs baseline: 4.1365x; 4.1365x over previous
import functools

import jax
import jax.numpy as jnp
from jax import lax
from jax.experimental import pallas as pl
from jax.experimental.pallas import tpu as pltpu
from jax.experimental.pallas import tpu_sc as plsc

N = 10000
E = 320000
NC = 2
NS = 16
KB = 80
NROW = 128
EPAD = NC * NS * KB * NROW
NACC = N + 16
STRIPE = 624
TAIL0 = NS * STRIPE
TAILN = N - TAIL0
EPC = E // NC
RB = 400
GRID = N // RB

f32 = jnp.float32


def _stripe_copy(src, dst, sid, src_off=0, dst_off=0):
    r0 = pl.multiple_of(sid * STRIPE, 8)
    pltpu.sync_copy(src.at[pl.ds(src_off + r0, STRIPE)],
                    dst.at[pl.ds(dst_off + r0, STRIPE)])

    @pl.when(sid == NS - 1)
    def _():
        pltpu.sync_copy(src.at[pl.ds(src_off + TAIL0, TAILN)],
                        dst.at[pl.ds(dst_off + TAIL0, TAILN)])


def _sc_count_body(sidx_hbm, ones_hbm, zeros_hbm, out_hbm, sv, ones_v, acc):
    cid = lax.axis_index("c")
    sid = lax.axis_index("s")
    _stripe_copy(zeros_hbm, acc, sid)
    wid = cid * NS + sid
    pltpu.sync_copy(sidx_hbm.at[wid], sv)
    pltpu.sync_copy(ones_hbm, ones_v)
    plsc.subcore_barrier()

    def body(j, carry):
        pltpu.sync_copy(ones_v, acc.at[sv.at[j]], add=True)
        return carry

    lax.fori_loop(0, NROW, body, 0)
    plsc.subcore_barrier()
    _stripe_copy(acc, out_hbm, sid, dst_off=cid * N)


def _sc_scatter_body(gidx_hbm, sidx_hbm, table_hbm, zeros_hbm, out_hbm,
                     gv, sv, rows, acc, sem):
    cid = lax.axis_index("c")
    sid = lax.axis_index("s")
    _stripe_copy(zeros_hbm, acc, sid)
    wid = cid * NS + sid
    pltpu.sync_copy(gidx_hbm.at[wid], gv)
    pltpu.sync_copy(sidx_hbm.at[wid], sv)
    plsc.subcore_barrier()

    def body(j, carry):
        pltpu.async_copy(table_hbm.at[gv.at[j]], rows, sem).wait()
        pltpu.sync_copy(rows, acc.at[sv.at[j]], add=True)
        return carry

    lax.fori_loop(0, NROW, body, 0)
    plsc.subcore_barrier()
    _stripe_copy(acc, out_hbm, sid, dst_off=cid * N)


@functools.lru_cache(maxsize=None)
def _sc_kernels():
    mesh = plsc.VectorSubcoreMesh(
        core_axis_name="c", subcore_axis_name="s",
        num_cores=NC, num_subcores=NS)
    count = pl.kernel(
        _sc_count_body,
        out_type=jax.ShapeDtypeStruct((2 * N, 128), f32),
        mesh=mesh,
        scratch_types=[
            pltpu.VMEM((NROW, KB), jnp.int32),
            pltpu.VMEM((KB, 128), f32),
            pltpu.VMEM_SHARED((NACC, 128), f32),
        ],
    )
    scatter = pl.kernel(
        _sc_scatter_body,
        out_type=jax.ShapeDtypeStruct((2 * N, 128), f32),
        mesh=mesh,
        scratch_types=[
            pltpu.VMEM((NROW, KB), jnp.int32),
            pltpu.VMEM((NROW, KB), jnp.int32),
            pltpu.VMEM((KB, 128), f32),
            pltpu.VMEM_SHARED((NACC, 128), f32),
            pltpu.SemaphoreType.DMA,
        ],
    )
    return count, scatter


def _row_spec(width):
    return pl.BlockSpec((RB, width), lambda i: (i, 0))


def _part_specs(width):
    return (pl.BlockSpec((RB, width), lambda i: (i, 0)),
            pl.BlockSpec((RB, width), lambda i: (i + GRID, 0)))


def _full_spec(shape):
    return pl.BlockSpec(shape, lambda i: tuple(0 for _ in shape))


def _prep_body(x_ref, dd0_ref, dd1_ref, sg0_ref, sg1_ref,
               xs_ref, dinv_ref, degs_ref):
    dd = dd0_ref[...][:, :1] + dd1_ref[...][:, :1] + 1.0
    dinv = lax.rsqrt(dd)
    dinv_ref[...] = jnp.broadcast_to(dinv, (RB, 16))
    sg = sg0_ref[...][:, :1] + sg1_ref[...][:, :1]
    degs_ref[...] = jnp.broadcast_to(sg, (RB, 16))
    xs_ref[...] = dinv * x_ref[...]


_prep = pl.pallas_call(
    _prep_body,
    grid=(GRID,),
    in_specs=[_row_spec(128), *_part_specs(128), *_part_specs(128)],
    out_specs=[_row_spec(128), _row_spec(16), _row_spec(16)],
    out_shape=[jax.ShapeDtypeStruct((N, 128), f32),
               jax.ShapeDtypeStruct((N, 16), f32),
               jax.ShapeDtypeStruct((N, 16), f32)],
)


def _layer1_body(s0_ref, s1_ref, xs_ref, dinv_ref, w_ref, b_ref,
                 hlo_ref, hhi_ref):
    g = dinv_ref[...][:, :1] * (s0_ref[...] + s1_ref[...] + xs_ref[...])
    h = jnp.dot(g, w_ref[...], preferred_element_type=f32) + b_ref[...]
    h = jnp.maximum(h, 0.0)
    hlo_ref[...] = h[:, :128]
    hhi_ref[...] = h[:, 128:]


_layer1 = pl.pallas_call(
    _layer1_body,
    grid=(GRID,),
    in_specs=[*_part_specs(128), _row_spec(128), _row_spec(16),
              _full_spec((128, 256)), _full_spec((1, 256))],
    out_specs=[_row_spec(128), _row_spec(128)],
    out_shape=[jax.ShapeDtypeStruct((N, 128), f32),
               jax.ShapeDtypeStruct((N, 128), f32)],
)


def _sample_body(nl0_ref, nl1_ref, nh0_ref, nh1_ref, hlo_ref, hhi_ref,
                 degs_ref, dinv_ref, slo_ref, shi_ref, xlo_ref, xhi_ref):
    deg = degs_ref[...]
    inv = (1.0 / jnp.maximum(deg, 1.0))[:, :1]
    pred = deg[:, :1] > 0.0
    slo = jnp.where(pred, (nl0_ref[...] + nl1_ref[...]) * inv, hlo_ref[...])
    shi = jnp.where(pred, (nh0_ref[...] + nh1_ref[...]) * inv, hhi_ref[...])
    slo_ref[...] = slo
    shi_ref[...] = shi
    dinv = dinv_ref[...][:, :1]
    xlo_ref[...] = dinv * slo
    xhi_ref[...] = dinv * shi


_sample = pl.pallas_call(
    _sample_body,
    grid=(GRID,),
    in_specs=[*_part_specs(128), *_part_specs(128), _row_spec(128),
              _row_spec(128), _row_spec(16), _row_spec(16)],
    out_specs=[_row_spec(128)] * 4,
    out_shape=[jax.ShapeDtypeStruct((N, 128), f32)] * 4,
)


def _layer23_body(sl0_ref, sl1_ref, sh0_ref, sh1_ref, xlo_ref, xhi_ref,
                  dinv_ref, w_ref, b_ref, hlo_ref, hhi_ref, ylo_ref, yhi_ref):
    dinv = dinv_ref[...][:, :1]
    glo = dinv * (sl0_ref[...] + sl1_ref[...] + xlo_ref[...])
    ghi = dinv * (sh0_ref[...] + sh1_ref[...] + xhi_ref[...])
    w = w_ref[...]
    h = (jnp.dot(glo, w[:128, :], preferred_element_type=f32)
         + jnp.dot(ghi, w[128:, :], preferred_element_type=f32) + b_ref[...])
    h = jnp.maximum(h, 0.0)
    hlo = h[:, :128]
    hhi = h[:, 128:]
    hlo_ref[...] = hlo
    hhi_ref[...] = hhi
    ylo_ref[...] = dinv * hlo
    yhi_ref[...] = dinv * hhi


_layer2 = pl.pallas_call(
    _layer23_body,
    grid=(GRID,),
    in_specs=[*_part_specs(128), *_part_specs(128), _row_spec(128),
              _row_spec(128), _row_spec(16),
              _full_spec((256, 256)), _full_spec((1, 256))],
    out_specs=[_row_spec(128)] * 4,
    out_shape=[jax.ShapeDtypeStruct((N, 128), f32)] * 4,
)


def _final_body(sl0_ref, sl1_ref, sh0_ref, sh1_ref, xlo_ref, xhi_ref,
                dinv_ref, w_ref, b_ref, h1lo_ref, h1hi_ref, h2lo_ref,
                h2hi_ref, jk_ref, fcw_ref, fcb_ref, out_ref):
    dinv = dinv_ref[...][:, :1]
    glo = dinv * (sl0_ref[...] + sl1_ref[...] + xlo_ref[...])
    ghi = dinv * (sh0_ref[...] + sh1_ref[...] + xhi_ref[...])
    w = w_ref[...]
    h3 = (jnp.dot(glo, w[:128, :], preferred_element_type=f32)
          + jnp.dot(ghi, w[128:, :], preferred_element_type=f32) + b_ref[...])
    h3 = jnp.maximum(h3, 0.0)
    e = jnp.exp(jk_ref[...])
    s = jnp.sum(e)
    w0 = e[0, 0] / s
    w1 = e[0, 1] / s
    w2 = e[0, 2] / s
    agg_lo = w0 * h1lo_ref[...] + w1 * h2lo_ref[...] + w2 * h3[:, :128]
    agg_hi = w0 * h1hi_ref[...] + w1 * h2hi_ref[...] + w2 * h3[:, 128:]
    fcw = fcw_ref[...]
    out_ref[...] = (jnp.dot(agg_lo, fcw[:128, :], preferred_element_type=f32)
                    + jnp.dot(agg_hi, fcw[128:, :], preferred_element_type=f32)
                    + fcb_ref[...])


_final = pl.pallas_call(
    _final_body,
    grid=(GRID,),
    in_specs=[*_part_specs(128), *_part_specs(128), _row_spec(128),
              _row_spec(128), _row_spec(16),
              _full_spec((256, 256)), _full_spec((1, 256)),
              _row_spec(128), _row_spec(128), _row_spec(128), _row_spec(128),
              _full_spec((1, 128)), _full_spec((256, 128)),
              _full_spec((1, 128))],
    out_specs=[_row_spec(128)],
    out_shape=[jax.ShapeDtypeStruct((N, 128), f32)],
)


def kernel(x, edge_index, W1, b1, W2, b2, W3, b3, jk, fcW, fcb):
    _sc_count, _sc_scatter = _sc_kernels()
    npad = EPAD - E
    def _pad3(idx, fill):
        return jnp.concatenate(
            [idx, jnp.full((npad,), fill, jnp.int32)]).reshape(NC * NS, NROW, KB)
    src_g = _pad3(edge_index[0], 0)
    src_s = _pad3(edge_index[0], N)
    dst_g = _pad3(edge_index[1], 0)
    dst_s = _pad3(edge_index[1], N)
    zeros128 = jnp.zeros((N, 128), f32)
    ones128 = jnp.ones((KB, 128), f32)

    degd = _sc_count(dst_s, ones128, zeros128)
    degs = _sc_count(src_s, ones128, zeros128)
    xs1, dinv16, degs16 = _prep(x, degd, degd, degs, degs)

    s1 = _sc_scatter(src_g, dst_s, xs1, zeros128)
    h1lo, h1hi = _layer1(s1, s1, xs1, dinv16, W1, b1.reshape(1, -1))

    nlo = _sc_scatter(dst_g, src_s, h1lo, zeros128)
    nhi = _sc_scatter(dst_g, src_s, h1hi, zeros128)
    h1slo, h1shi, xs2lo, xs2hi = _sample(
        nlo, nlo, nhi, nhi, h1lo, h1hi, degs16, dinv16)

    s2lo = _sc_scatter(src_g, dst_s, xs2lo, zeros128)
    s2hi = _sc_scatter(src_g, dst_s, xs2hi, zeros128)
    h2lo, h2hi, xs3lo, xs3hi = _layer2(
        s2lo, s2lo, s2hi, s2hi, xs2lo, xs2hi, dinv16, W2, b2.reshape(1, -1))

    s3lo = _sc_scatter(src_g, dst_s, xs3lo, zeros128)
    s3hi = _sc_scatter(src_g, dst_s, xs3hi, zeros128)

    jkpad = jnp.full((1, 128), -jnp.inf, f32).at[0, :3].set(jk)
    (out,) = _final(
        s3lo, s3lo, s3hi, s3hi, xs3lo, xs3hi, dinv16, W3, b3.reshape(1, -1),
        h1slo, h1shi, h2lo, h2hi, jkpad, fcW, fcb.reshape(1, -1))
    return out


# speedup vs baseline: 4.7603x; 1.1508x over previous
import functools

import jax
import jax.numpy as jnp
from jax import lax
from jax.experimental import pallas as pl
from jax.experimental.pallas import tpu as pltpu
from jax.experimental.pallas import tpu_sc as plsc

N = 10000
E = 320000
NC = 2
NS = 16
KB = 128
NROW = 80
HROW = 40
EPAD = NC * NS * KB * NROW
NACC = N + 16
STRIPE = 624
TAIL0 = NS * STRIPE
TAILN = N - TAIL0
EPC = E // NC
RB = 400
GRID = N // RB

f32 = jnp.float32


def _stripe_copy(src, dst, sid, src_off=0, dst_off=0):
    r0 = pl.multiple_of(sid * STRIPE, 8)
    pltpu.sync_copy(src.at[pl.ds(src_off + r0, STRIPE)],
                    dst.at[pl.ds(dst_off + r0, STRIPE)])

    @pl.when(sid == NS - 1)
    def _():
        pltpu.sync_copy(src.at[pl.ds(src_off + TAIL0, TAILN)],
                        dst.at[pl.ds(dst_off + TAIL0, TAILN)])


def _sc_count_body(sidx_hbm, ones_hbm, zeros_hbm, out_hbm, sv, ones_v, acc):
    cid = lax.axis_index("c")
    sid = lax.axis_index("s")
    _stripe_copy(zeros_hbm, acc, sid)
    wid = cid * NS + sid
    pltpu.sync_copy(sidx_hbm.at[wid], sv)
    pltpu.sync_copy(ones_hbm, ones_v)
    plsc.subcore_barrier()

    def body(j, carry):
        pltpu.sync_copy(ones_v, acc.at[sv.at[j]], add=True)
        return carry

    lax.fori_loop(0, NROW, body, 0)
    plsc.subcore_barrier()
    _stripe_copy(acc, out_hbm, sid, dst_off=cid * N)


def _sc_scatter_body(gidx_hbm, sidx_hbm, table_hbm, zeros_hbm, out_hbm,
                     gv, sv, rows_a, rows_b, acc, sem_a, sem_b):
    cid = lax.axis_index("c")
    sid = lax.axis_index("s")
    _stripe_copy(zeros_hbm, acc, sid)
    wid = cid * NS + sid
    plsc.subcore_barrier()

    for p in range(NROW // HROW):
        pltpu.sync_copy(gidx_hbm.at[wid, pl.ds(p * HROW, HROW)], gv)
        pltpu.sync_copy(sidx_hbm.at[wid, pl.ds(p * HROW, HROW)], sv)
        pltpu.async_copy(table_hbm.at[gv.at[0]], rows_a, sem_a)
        pltpu.async_copy(table_hbm.at[gv.at[1]], rows_b, sem_b)

        def body(jj, carry):
            j0 = jj * 2
            pltpu.make_async_copy(table_hbm.at[gv.at[j0]], rows_a, sem_a).wait()
            pltpu.sync_copy(rows_a, acc.at[sv.at[j0]], add=True)
            pltpu.async_copy(table_hbm.at[gv.at[j0 + 2]], rows_a, sem_a)
            pltpu.make_async_copy(
                table_hbm.at[gv.at[j0 + 1]], rows_b, sem_b).wait()
            pltpu.sync_copy(rows_b, acc.at[sv.at[j0 + 1]], add=True)
            pltpu.async_copy(table_hbm.at[gv.at[j0 + 3]], rows_b, sem_b)
            return carry

        lax.fori_loop(0, HROW // 2 - 1, body, 0)
        j0 = HROW - 2
        pltpu.make_async_copy(table_hbm.at[gv.at[j0]], rows_a, sem_a).wait()
        pltpu.sync_copy(rows_a, acc.at[sv.at[j0]], add=True)
        pltpu.make_async_copy(table_hbm.at[gv.at[j0 + 1]], rows_b, sem_b).wait()
        pltpu.sync_copy(rows_b, acc.at[sv.at[j0 + 1]], add=True)

    plsc.subcore_barrier()
    _stripe_copy(acc, out_hbm, sid, dst_off=cid * N)


@functools.lru_cache(maxsize=None)
def _sc_kernels():
    mesh = plsc.VectorSubcoreMesh(
        core_axis_name="c", subcore_axis_name="s",
        num_cores=NC, num_subcores=NS)
    count = pl.kernel(
        _sc_count_body,
        out_type=jax.ShapeDtypeStruct((2 * N, 128), f32),
        mesh=mesh,
        scratch_types=[
            pltpu.VMEM((NROW, KB), jnp.int32),
            pltpu.VMEM((KB, 128), f32),
            pltpu.VMEM_SHARED((NACC, 128), f32),
        ],
    )
    scatter = pl.kernel(
        _sc_scatter_body,
        out_type=jax.ShapeDtypeStruct((2 * N, 128), f32),
        mesh=mesh,
        scratch_types=[
            pltpu.VMEM((HROW, KB), jnp.int32),
            pltpu.VMEM((HROW, KB), jnp.int32),
            pltpu.VMEM((KB, 128), f32),
            pltpu.VMEM((KB, 128), f32),
            pltpu.VMEM_SHARED((NACC, 128), f32),
            pltpu.SemaphoreType.DMA,
            pltpu.SemaphoreType.DMA,
        ],
    )
    return count, scatter


def _row_spec(width):
    return pl.BlockSpec((RB, width), lambda i: (i, 0))


def _part_specs(width):
    return (pl.BlockSpec((RB, width), lambda i: (i, 0)),
            pl.BlockSpec((RB, width), lambda i: (i + GRID, 0)))


def _full_spec(shape):
    return pl.BlockSpec(shape, lambda i: tuple(0 for _ in shape))


def _prep_body(x_ref, dd0_ref, dd1_ref, sg0_ref, sg1_ref,
               xs_ref, dinv_ref, degs_ref):
    dd = dd0_ref[...][:, :1] + dd1_ref[...][:, :1] + 1.0
    dinv = lax.rsqrt(dd)
    dinv_ref[...] = jnp.broadcast_to(dinv, (RB, 16))
    sg = sg0_ref[...][:, :1] + sg1_ref[...][:, :1]
    degs_ref[...] = jnp.broadcast_to(sg, (RB, 16))
    xs_ref[...] = dinv * x_ref[...]


_prep = pl.pallas_call(
    _prep_body,
    grid=(GRID,),
    in_specs=[_row_spec(128), *_part_specs(128), *_part_specs(128)],
    out_specs=[_row_spec(128), _row_spec(16), _row_spec(16)],
    out_shape=[jax.ShapeDtypeStruct((N, 128), f32),
               jax.ShapeDtypeStruct((N, 16), f32),
               jax.ShapeDtypeStruct((N, 16), f32)],
)


def _layer1_body(s0_ref, s1_ref, xs_ref, dinv_ref, w_ref, b_ref,
                 hlo_ref, hhi_ref):
    g = dinv_ref[...][:, :1] * (s0_ref[...] + s1_ref[...] + xs_ref[...])
    h = jnp.dot(g, w_ref[...], preferred_element_type=f32) + b_ref[...]
    h = jnp.maximum(h, 0.0)
    hlo_ref[...] = h[:, :128]
    hhi_ref[...] = h[:, 128:]


_layer1 = pl.pallas_call(
    _layer1_body,
    grid=(GRID,),
    in_specs=[*_part_specs(128), _row_spec(128), _row_spec(16),
              _full_spec((128, 256)), _full_spec((1, 256))],
    out_specs=[_row_spec(128), _row_spec(128)],
    out_shape=[jax.ShapeDtypeStruct((N, 128), f32),
               jax.ShapeDtypeStruct((N, 128), f32)],
)


def _sample_body(nl0_ref, nl1_ref, nh0_ref, nh1_ref, hlo_ref, hhi_ref,
                 degs_ref, dinv_ref, slo_ref, shi_ref, xlo_ref, xhi_ref):
    deg = degs_ref[...]
    inv = (1.0 / jnp.maximum(deg, 1.0))[:, :1]
    pred = deg[:, :1] > 0.0
    slo = jnp.where(pred, (nl0_ref[...] + nl1_ref[...]) * inv, hlo_ref[...])
    shi = jnp.where(pred, (nh0_ref[...] + nh1_ref[...]) * inv, hhi_ref[...])
    slo_ref[...] = slo
    shi_ref[...] = shi
    dinv = dinv_ref[...][:, :1]
    xlo_ref[...] = dinv * slo
    xhi_ref[...] = dinv * shi


_sample = pl.pallas_call(
    _sample_body,
    grid=(GRID,),
    in_specs=[*_part_specs(128), *_part_specs(128), _row_spec(128),
              _row_spec(128), _row_spec(16), _row_spec(16)],
    out_specs=[_row_spec(128)] * 4,
    out_shape=[jax.ShapeDtypeStruct((N, 128), f32)] * 4,
)


def _layer23_body(sl0_ref, sl1_ref, sh0_ref, sh1_ref, xlo_ref, xhi_ref,
                  dinv_ref, w_ref, b_ref, hlo_ref, hhi_ref, ylo_ref, yhi_ref):
    dinv = dinv_ref[...][:, :1]
    glo = dinv * (sl0_ref[...] + sl1_ref[...] + xlo_ref[...])
    ghi = dinv * (sh0_ref[...] + sh1_ref[...] + xhi_ref[...])
    w = w_ref[...]
    h = (jnp.dot(glo, w[:128, :], preferred_element_type=f32)
         + jnp.dot(ghi, w[128:, :], preferred_element_type=f32) + b_ref[...])
    h = jnp.maximum(h, 0.0)
    hlo = h[:, :128]
    hhi = h[:, 128:]
    hlo_ref[...] = hlo
    hhi_ref[...] = hhi
    ylo_ref[...] = dinv * hlo
    yhi_ref[...] = dinv * hhi


_layer2 = pl.pallas_call(
    _layer23_body,
    grid=(GRID,),
    in_specs=[*_part_specs(128), *_part_specs(128), _row_spec(128),
              _row_spec(128), _row_spec(16),
              _full_spec((256, 256)), _full_spec((1, 256))],
    out_specs=[_row_spec(128)] * 4,
    out_shape=[jax.ShapeDtypeStruct((N, 128), f32)] * 4,
)


def _final_body(sl0_ref, sl1_ref, sh0_ref, sh1_ref, xlo_ref, xhi_ref,
                dinv_ref, w_ref, b_ref, h1lo_ref, h1hi_ref, h2lo_ref,
                h2hi_ref, jk_ref, fcw_ref, fcb_ref, out_ref):
    dinv = dinv_ref[...][:, :1]
    glo = dinv * (sl0_ref[...] + sl1_ref[...] + xlo_ref[...])
    ghi = dinv * (sh0_ref[...] + sh1_ref[...] + xhi_ref[...])
    w = w_ref[...]
    h3 = (jnp.dot(glo, w[:128, :], preferred_element_type=f32)
          + jnp.dot(ghi, w[128:, :], preferred_element_type=f32) + b_ref[...])
    h3 = jnp.maximum(h3, 0.0)
    e = jnp.exp(jk_ref[...])
    s = jnp.sum(e)
    w0 = e[0, 0] / s
    w1 = e[0, 1] / s
    w2 = e[0, 2] / s
    agg_lo = w0 * h1lo_ref[...] + w1 * h2lo_ref[...] + w2 * h3[:, :128]
    agg_hi = w0 * h1hi_ref[...] + w1 * h2hi_ref[...] + w2 * h3[:, 128:]
    fcw = fcw_ref[...]
    out_ref[...] = (jnp.dot(agg_lo, fcw[:128, :], preferred_element_type=f32)
                    + jnp.dot(agg_hi, fcw[128:, :], preferred_element_type=f32)
                    + fcb_ref[...])


_final = pl.pallas_call(
    _final_body,
    grid=(GRID,),
    in_specs=[*_part_specs(128), *_part_specs(128), _row_spec(128),
              _row_spec(128), _row_spec(16),
              _full_spec((256, 256)), _full_spec((1, 256)),
              _row_spec(128), _row_spec(128), _row_spec(128), _row_spec(128),
              _full_spec((1, 128)), _full_spec((256, 128)),
              _full_spec((1, 128))],
    out_specs=[_row_spec(128)],
    out_shape=[jax.ShapeDtypeStruct((N, 128), f32)],
)


def kernel(x, edge_index, W1, b1, W2, b2, W3, b3, jk, fcW, fcb):
    _sc_count, _sc_scatter = _sc_kernels()
    npad = EPAD - E
    def _pad3(idx, fill):
        return jnp.concatenate(
            [idx, jnp.full((npad,), fill, jnp.int32)]).reshape(NC * NS, NROW, KB)
    src_g = _pad3(edge_index[0], 0)
    src_s = _pad3(edge_index[0], N)
    dst_g = _pad3(edge_index[1], 0)
    dst_s = _pad3(edge_index[1], N)
    zeros128 = jnp.zeros((N, 128), f32)
    ones128 = jnp.ones((KB, 128), f32)

    degd = _sc_count(dst_s, ones128, zeros128)
    degs = _sc_count(src_s, ones128, zeros128)
    xs1, dinv16, degs16 = _prep(x, degd, degd, degs, degs)

    s1 = _sc_scatter(src_g, dst_s, xs1, zeros128)
    h1lo, h1hi = _layer1(s1, s1, xs1, dinv16, W1, b1.reshape(1, -1))

    nlo = _sc_scatter(dst_g, src_s, h1lo, zeros128)
    nhi = _sc_scatter(dst_g, src_s, h1hi, zeros128)
    h1slo, h1shi, xs2lo, xs2hi = _sample(
        nlo, nlo, nhi, nhi, h1lo, h1hi, degs16, dinv16)

    s2lo = _sc_scatter(src_g, dst_s, xs2lo, zeros128)
    s2hi = _sc_scatter(src_g, dst_s, xs2hi, zeros128)
    h2lo, h2hi, xs3lo, xs3hi = _layer2(
        s2lo, s2lo, s2hi, s2hi, xs2lo, xs2hi, dinv16, W2, b2.reshape(1, -1))

    s3lo = _sc_scatter(src_g, dst_s, xs3lo, zeros128)
    s3hi = _sc_scatter(src_g, dst_s, xs3hi, zeros128)

    jkpad = jnp.full((1, 128), -jnp.inf, f32).at[0, :3].set(jk)
    (out,) = _final(
        s3lo, s3lo, s3hi, s3hi, xs3lo, xs3hi, dinv16, W3, b3.reshape(1, -1),
        h1slo, h1shi, h2lo, h2hi, jkpad, fcW, fcb.reshape(1, -1))
    return out
